# baseline (speedup 1.0000x reference)
import functools

import jax
import jax.numpy as jnp
from jax import lax
from jax.experimental import pallas as pl
from jax.experimental.pallas import tpu as pltpu
from jax.experimental.pallas import tpu_sc as plsc

N = 10000
E = 320000
F = 128
H = 128
C = 40
CP = 48
HH = 64

NC = 2
NS = 16
NW = NC * NS
EW = E // NW
K = 80
NCHUNK = EW // K
RPT = 624
TAIL0 = NS * RPT
TAIL = N - TAIL0


@functools.lru_cache(maxsize=None)
def _make_agg(D, with_deg, NBUF, P):
    mesh = plsc.VectorSubcoreMesh(
        core_axis_name="c", subcore_axis_name="s",
        num_cores=NC, num_subcores=NS)
    parts = [jax.ShapeDtypeStruct((NC, N, D), jnp.float32)] * P
    if with_deg:
        parts.append(jax.ShapeDtypeStruct((NC, N, 16), jnp.float32))
    out_type = tuple(parts) if len(parts) > 1 else parts[0]
    scratch = [
        pltpu.VMEM((NCHUNK, K), jnp.int32),
        pltpu.VMEM((NCHUNK, K), jnp.int32),
    ] + [pltpu.VMEM((K, D), jnp.float32) for _ in range(NBUF)] + [
        pltpu.VMEM_SHARED((N, D), jnp.float32),
    ] + [pltpu.SemaphoreType.DMA for _ in range(NBUF + 1)]
    if with_deg:
        scratch += [
            pltpu.VMEM((K, 16), jnp.float32),
            pltpu.VMEM_SHARED((N, 16), jnp.float32),
        ]

    def body(*refs):
        tables = refs[:P]
        srcs, dsts, zrows = refs[P], refs[P + 1], refs[P + 2]
        nin = P + 3
        if with_deg:
            zdeg, ones_h = refs[P + 3], refs[P + 4]
            nin = P + 5
        outs = refs[nin:nin + P]
        deg_out = refs[nin + P] if with_deg else None
        rest = refs[nin + P + (1 if with_deg else 0):]
        src_v, dst_v = rest[0], rest[1]
        bufs = rest[2:2 + NBUF]
        acc = rest[2 + NBUF]
        sems = rest[3 + NBUF:3 + 2 * NBUF]
        dsem = rest[3 + 2 * NBUF]
        if with_deg:
            ones_v, dacc = rest[4 + 2 * NBUF], rest[5 + 2 * NBUF]

        c = lax.axis_index("c")
        s = lax.axis_index("s")
        wid = s * NC + c
        r0 = s * RPT
        last_tile = s == NS - 1

        pltpu.sync_copy(srcs.at[wid], src_v)
        pltpu.sync_copy(dsts.at[wid], dst_v)
        if with_deg:
            pltpu.sync_copy(ones_h, ones_v)
            pltpu.sync_copy(zdeg.at[pl.ds(r0, RPT)], dacc.at[pl.ds(r0, RPT)])

            @pl.when(last_tile)
            def _():
                pltpu.sync_copy(zdeg.at[pl.ds(TAIL0, TAIL)],
                                dacc.at[pl.ds(TAIL0, TAIL)])

        for p in range(P):
            y, out = tables[p], outs[p]
            deg_here = with_deg and p == 0

            pltpu.sync_copy(zrows.at[pl.ds(r0, RPT)], acc.at[pl.ds(r0, RPT)])

            @pl.when(last_tile)
            def _():
                pltpu.sync_copy(zrows.at[pl.ds(TAIL0, TAIL)],
                                acc.at[pl.ds(TAIL0, TAIL)])

            plsc.subcore_barrier()

            def gather(i, b):
                pltpu.async_copy(y.at[src_v.at[i]], bufs[b], sems[b])

            def slot(i, b):
                pltpu.make_async_copy(y.at[src_v.at[i]], bufs[b],
                                      sems[b]).wait()
                pltpu.sync_copy(bufs[b], acc.at[dst_v.at[i]], add=True)
                if deg_here:
                    pltpu.async_copy(ones_v, dacc.at[dst_v.at[i]], dsem,
                                     add=True)

                @pl.when(i + NBUF < NCHUNK)
                def _():
                    gather(i + NBUF, b)

                if deg_here:
                    pltpu.make_async_copy(ones_v, dacc.at[dst_v.at[i]],
                                          dsem).wait()

            for b in range(NBUF):
                gather(b, b)

            def turn(t, carry):
                for b in range(NBUF):
                    slot(t * NBUF + b, b)
                return carry

            lax.fori_loop(0, NCHUNK // NBUF, turn, 0)
            for r in range(NCHUNK % NBUF):
                slot(NBUF * (NCHUNK // NBUF) + r, r)
            plsc.subcore_barrier()

            pltpu.sync_copy(acc.at[pl.ds(r0, RPT)], out.at[c, pl.ds(r0, RPT)])
            if deg_here:
                pltpu.sync_copy(dacc.at[pl.ds(r0, RPT)],
                                deg_out.at[c, pl.ds(r0, RPT)])

            @pl.when(last_tile)
            def _flush_tail():
                pltpu.sync_copy(acc.at[pl.ds(TAIL0, TAIL)],
                                out.at[c, pl.ds(TAIL0, TAIL)])
                if deg_here:
                    pltpu.sync_copy(dacc.at[pl.ds(TAIL0, TAIL)],
                                    deg_out.at[c, pl.ds(TAIL0, TAIL)])

    return pl.kernel(body, out_type=out_type, mesh=mesh,
                     scratch_types=scratch,
                     compiler_params=pltpu.CompilerParams(
                         use_tc_tiling_on_sc=False))


_BM = 1000


def _tc1_body(x, wla, wlb, wr, bl, y1a, y1b, r1b):
    xv = x[...]
    y1a[...] = jnp.dot(xv, wla[...], preferred_element_type=jnp.float32)
    y1b[...] = jnp.dot(xv, wlb[...], preferred_element_type=jnp.float32)
    r1b[...] = jnp.dot(xv, wr[...], preferred_element_type=jnp.float32) + bl[...]


def _tc2_body(a1a, a1b, degp, r1b, wl2, wr2, bl2, y2, r2b, deg):
    agg = jnp.concatenate([a1a[0] + a1a[1], a1b[0] + a1b[1]], axis=1)
    d = jnp.maximum(degp[0] + degp[1], 1.0)
    pre = agg / d[:, 0:1] + r1b[...]
    h = jnp.where(pre > 0, pre, jnp.exp(jnp.minimum(pre, 0.0)) - 1.0)
    y2[...] = jnp.dot(h, wl2[...], preferred_element_type=jnp.float32)
    r2b[...] = jnp.dot(h, wr2[...], preferred_element_type=jnp.float32) + bl2[...]
    deg[...] = d


def _tc3_body(a2, deg, r2b, out):
    out[...] = (a2[0, :, :C] + a2[1, :, :C]) / deg[:, 0:1] + r2b[...]


def kernel(x, edge_index, Wl1, bl1, Wr1, Wl2, bl2, Wr2, Q, epoch):
    src3 = edge_index[0].reshape(NW, NCHUNK, K)
    dst3 = edge_index[1].reshape(NW, NCHUNK, K)
    f32 = jnp.float32

    nblk = N // _BM
    row_spec = lambda w: pl.BlockSpec((_BM, w), lambda i: (i, 0))
    full = lambda shape: pl.BlockSpec(shape, lambda i: tuple(0 for _ in shape))
    part_spec = lambda w: pl.BlockSpec((NC, _BM, w), lambda i: (0, i, 0))

    y1a, y1b, r1b = pl.pallas_call(
        _tc1_body,
        grid=(nblk,),
        in_specs=[row_spec(F), full((F, HH)), full((F, HH)), full((F, H)),
                  full((1, H))],
        out_specs=[row_spec(HH), row_spec(HH), row_spec(H)],
        out_shape=[jax.ShapeDtypeStruct((N, HH), f32),
                   jax.ShapeDtypeStruct((N, HH), f32),
                   jax.ShapeDtypeStruct((N, H), f32)],
    )(x, Wl1[:, :HH], Wl1[:, HH:], Wr1, bl1.reshape(1, H))

    zrows = jnp.zeros((N, HH), f32)
    zdeg = jnp.zeros((N, 16), f32)
    ones_h = jnp.ones((K, 16), f32)
    a1a, a1b, degp = _make_agg(HH, True, 8, 2)(
        y1a, y1b, src3, dst3, zrows, zdeg, ones_h)

    Wl2p = jnp.pad(Wl2, ((0, 0), (0, CP - C)))
    y2, r2b, deg = pl.pallas_call(
        _tc2_body,
        grid=(nblk,),
        in_specs=[part_spec(HH), part_spec(HH), part_spec(16), row_spec(H),
                  full((H, CP)), full((H, C)), full((1, C))],
        out_specs=[row_spec(CP), row_spec(C), row_spec(16)],
        out_shape=[jax.ShapeDtypeStruct((N, CP), f32),
                   jax.ShapeDtypeStruct((N, C), f32),
                   jax.ShapeDtypeStruct((N, 16), f32)],
    )(a1a, a1b, degp, r1b, Wl2p, Wr2, bl2.reshape(1, C))

    zrows2 = jnp.zeros((N, CP), f32)
    a2 = _make_agg(CP, False, 6, 1)(y2, src3, dst3, zrows2)

    out = pl.pallas_call(
        _tc3_body,
        grid=(nblk,),
        in_specs=[part_spec(CP), row_spec(16), row_spec(C)],
        out_specs=row_spec(C),
        out_shape=jax.ShapeDtypeStruct((N, C), f32),
    )(a2, deg, r2b)

    return (out, Q)


# speedup vs baseline: 1.0934x; 1.0934x over previous
import functools

import jax
import jax.numpy as jnp
from jax import lax
from jax.experimental import pallas as pl
from jax.experimental.pallas import tpu as pltpu
from jax.experimental.pallas import tpu_sc as plsc

N = 10000
E = 320000
F = 128
H = 128
C = 40
CP = 48

NC = 2
NS = 16
NW = NC * NS
EW = E // NW
K = 40
NCHUNK = EW // K
DW = 8
RPT = 624
TAIL0 = NS * RPT
TAIL = N - TAIL0


@functools.lru_cache(maxsize=None)
def _make_agg(D, with_deg, NBUF, P):
    mesh = plsc.VectorSubcoreMesh(
        core_axis_name="c", subcore_axis_name="s",
        num_cores=NC, num_subcores=NS)
    parts = [jax.ShapeDtypeStruct((NC, N, D), jnp.float32)] * P
    if with_deg:
        parts.append(jax.ShapeDtypeStruct((NC, N, DW), jnp.float32))
    out_type = tuple(parts) if len(parts) > 1 else parts[0]
    scratch = [
        pltpu.VMEM((NCHUNK, K), jnp.int32),
        pltpu.VMEM((NCHUNK, K), jnp.int32),
    ] + [pltpu.VMEM((K, D), jnp.float32) for _ in range(NBUF)] + [
        pltpu.VMEM_SHARED((N, D), jnp.float32),
    ] + [pltpu.SemaphoreType.DMA for _ in range(NBUF + 1)]
    if with_deg:
        scratch += [
            pltpu.VMEM((K, DW), jnp.float32),
            pltpu.VMEM_SHARED((N, DW), jnp.float32),
        ]

    def body(*refs):
        tables = refs[:P]
        srcs, dsts, zrows = refs[P], refs[P + 1], refs[P + 2]
        nin = P + 3
        if with_deg:
            zdeg, ones_h = refs[P + 3], refs[P + 4]
            nin = P + 5
        outs = refs[nin:nin + P]
        deg_out = refs[nin + P] if with_deg else None
        rest = refs[nin + P + (1 if with_deg else 0):]
        src_v, dst_v = rest[0], rest[1]
        bufs = rest[2:2 + NBUF]
        acc = rest[2 + NBUF]
        sems = rest[3 + NBUF:3 + 2 * NBUF]
        dsem = rest[3 + 2 * NBUF]
        if with_deg:
            ones_v, dacc = rest[4 + 2 * NBUF], rest[5 + 2 * NBUF]

        c = lax.axis_index("c")
        s = lax.axis_index("s")
        wid = s * NC + c
        r0 = s * RPT
        last_tile = s == NS - 1

        pltpu.sync_copy(srcs.at[wid], src_v)
        pltpu.sync_copy(dsts.at[wid], dst_v)
        if with_deg:
            pltpu.sync_copy(ones_h, ones_v)
            pltpu.sync_copy(zdeg.at[pl.ds(r0, RPT)], dacc.at[pl.ds(r0, RPT)])

            @pl.when(last_tile)
            def _():
                pltpu.sync_copy(zdeg.at[pl.ds(TAIL0, TAIL)],
                                dacc.at[pl.ds(TAIL0, TAIL)])

        for p in range(P):
            y, out = tables[p], outs[p]
            deg_here = with_deg and p == 0

            pltpu.sync_copy(zrows.at[pl.ds(r0, RPT)], acc.at[pl.ds(r0, RPT)])

            @pl.when(last_tile)
            def _():
                pltpu.sync_copy(zrows.at[pl.ds(TAIL0, TAIL)],
                                acc.at[pl.ds(TAIL0, TAIL)])

            plsc.subcore_barrier()

            def gather(i, b):
                pltpu.async_copy(y.at[src_v.at[i]], bufs[b], sems[b])

            def slot(i, b):
                pltpu.make_async_copy(y.at[src_v.at[i]], bufs[b],
                                      sems[b]).wait()
                pltpu.sync_copy(bufs[b], acc.at[dst_v.at[i]], add=True)
                if deg_here:
                    pltpu.async_copy(ones_v, dacc.at[dst_v.at[i]], dsem,
                                     add=True)

                @pl.when(i + NBUF < NCHUNK)
                def _():
                    gather(i + NBUF, b)

                if deg_here:
                    pltpu.make_async_copy(ones_v, dacc.at[dst_v.at[i]],
                                          dsem).wait()

            for b in range(NBUF):
                gather(b, b)

            def turn(t, carry):
                for b in range(NBUF):
                    slot(t * NBUF + b, b)
                return carry

            lax.fori_loop(0, NCHUNK // NBUF, turn, 0)
            for r in range(NCHUNK % NBUF):
                slot(NBUF * (NCHUNK // NBUF) + r, r)
            plsc.subcore_barrier()

            pltpu.sync_copy(acc.at[pl.ds(r0, RPT)], out.at[c, pl.ds(r0, RPT)])
            if deg_here:
                pltpu.sync_copy(dacc.at[pl.ds(r0, RPT)],
                                deg_out.at[c, pl.ds(r0, RPT)])

            @pl.when(last_tile)
            def _flush_tail():
                pltpu.sync_copy(acc.at[pl.ds(TAIL0, TAIL)],
                                out.at[c, pl.ds(TAIL0, TAIL)])
                if deg_here:
                    pltpu.sync_copy(dacc.at[pl.ds(TAIL0, TAIL)],
                                    deg_out.at[c, pl.ds(TAIL0, TAIL)])

    return pl.kernel(body, out_type=out_type, mesh=mesh,
                     scratch_types=scratch,
                     compiler_params=pltpu.CompilerParams(
                         use_tc_tiling_on_sc=False))


_BM = 1000


def _tc1_body(x, wl, wr, bl, y1, r1b):
    xv = x[...]
    y1[...] = jnp.dot(xv, wl[...], preferred_element_type=jnp.float32)
    r1b[...] = jnp.dot(xv, wr[...], preferred_element_type=jnp.float32) + bl[...]


def _tc2_body(a1, degp, r1b, wl2, wr2, bl2, y2, r2b, deg):
    agg = a1[0] + a1[1]
    d = jnp.maximum(degp[0] + degp[1], 1.0)
    pre = agg / d[:, 0:1] + r1b[...]
    h = jnp.where(pre > 0, pre, jnp.exp(jnp.minimum(pre, 0.0)) - 1.0)
    y2[...] = jnp.dot(h, wl2[...], preferred_element_type=jnp.float32)
    r2b[...] = jnp.dot(h, wr2[...], preferred_element_type=jnp.float32) + bl2[...]
    deg[...] = d


def _tc3_body(a2, deg, r2b, out):
    out[...] = (a2[0, :, :C] + a2[1, :, :C]) / deg[:, 0:1] + r2b[...]


def kernel(x, edge_index, Wl1, bl1, Wr1, Wl2, bl2, Wr2, Q, epoch):
    src3 = edge_index[0].reshape(NW, NCHUNK, K)
    dst3 = edge_index[1].reshape(NW, NCHUNK, K)
    f32 = jnp.float32

    nblk = N // _BM
    row_spec = lambda w: pl.BlockSpec((_BM, w), lambda i: (i, 0))
    full = lambda shape: pl.BlockSpec(shape, lambda i: tuple(0 for _ in shape))
    part_spec = lambda w: pl.BlockSpec((NC, _BM, w), lambda i: (0, i, 0))

    y1, r1b = pl.pallas_call(
        _tc1_body,
        grid=(nblk,),
        in_specs=[row_spec(F), full((F, H)), full((F, H)), full((1, H))],
        out_specs=[row_spec(H), row_spec(H)],
        out_shape=[jax.ShapeDtypeStruct((N, H), f32),
                   jax.ShapeDtypeStruct((N, H), f32)],
    )(x, Wl1, Wr1, bl1.reshape(1, H))

    zrows = jnp.zeros((N, H), f32)
    zdeg = jnp.zeros((N, DW), f32)
    ones_h = jnp.ones((K, DW), f32)
    a1, degp = _make_agg(H, True, 5, 1)(y1, src3, dst3, zrows, zdeg, ones_h)

    Wl2p = jnp.pad(Wl2, ((0, 0), (0, CP - C)))
    y2, r2b, deg = pl.pallas_call(
        _tc2_body,
        grid=(nblk,),
        in_specs=[part_spec(H), part_spec(DW), row_spec(H),
                  full((H, CP)), full((H, C)), full((1, C))],
        out_specs=[row_spec(CP), row_spec(C), row_spec(DW)],
        out_shape=[jax.ShapeDtypeStruct((N, CP), f32),
                   jax.ShapeDtypeStruct((N, C), f32),
                   jax.ShapeDtypeStruct((N, DW), f32)],
    )(a1, degp, r1b, Wl2p, Wr2, bl2.reshape(1, C))

    zrows2 = jnp.zeros((N, CP), f32)
    a2 = _make_agg(CP, False, 12, 1)(y2, src3, dst3, zrows2)

    out = pl.pallas_call(
        _tc3_body,
        grid=(nblk,),
        in_specs=[part_spec(CP), row_spec(DW), row_spec(C)],
        out_specs=row_spec(C),
        out_shape=jax.ShapeDtypeStruct((N, C), f32),
    )(a2, deg, r2b)

    return (out, Q)


# speedup vs baseline: 1.1344x; 1.0375x over previous
import functools

import jax
import jax.numpy as jnp
from jax import lax
from jax.experimental import pallas as pl
from jax.experimental.pallas import tpu as pltpu
from jax.experimental.pallas import tpu_sc as plsc

N = 10000
E = 320000
F = 128
H = 128
C = 40
CP = 48

NC = 2
NS = 16
NW = NC * NS
EW = E // NW
K = 40
NCHUNK = EW // K
DW = 8
RPT = 624
TAIL0 = NS * RPT
TAIL = N - TAIL0


@functools.lru_cache(maxsize=None)
def _make_agg(D, with_deg, NBUF, P):
    mesh = plsc.VectorSubcoreMesh(
        core_axis_name="c", subcore_axis_name="s",
        num_cores=NC, num_subcores=NS)
    parts = [jax.ShapeDtypeStruct((NC, N, D), jnp.float32)] * P
    if with_deg:
        parts.append(jax.ShapeDtypeStruct((NC, N, DW), jnp.float32))
    out_type = tuple(parts) if len(parts) > 1 else parts[0]
    scratch = [
        pltpu.VMEM((NCHUNK, K), jnp.int32),
        pltpu.VMEM((NCHUNK, K), jnp.int32),
    ] + [pltpu.VMEM((K, D), jnp.float32) for _ in range(NBUF)] + [
        pltpu.VMEM_SHARED((N, D), jnp.float32),
    ] + [pltpu.SemaphoreType.DMA for _ in range(NBUF + 1)]
    if with_deg:
        scratch += [
            pltpu.VMEM((K, DW), jnp.float32),
            pltpu.VMEM_SHARED((N, DW), jnp.float32),
        ]

    def body(*refs):
        tables = refs[:P]
        edges, zrows = refs[P], refs[P + 1]
        nin = P + 2
        if with_deg:
            zdeg, ones_h = refs[P + 2], refs[P + 3]
            nin = P + 4
        outs = refs[nin:nin + P]
        deg_out = refs[nin + P] if with_deg else None
        rest = refs[nin + P + (1 if with_deg else 0):]
        src_v, dst_v = rest[0], rest[1]
        bufs = rest[2:2 + NBUF]
        acc = rest[2 + NBUF]
        sems = rest[3 + NBUF:3 + 2 * NBUF]
        dsem = rest[3 + 2 * NBUF]
        if with_deg:
            ones_v, dacc = rest[4 + 2 * NBUF], rest[5 + 2 * NBUF]

        c = lax.axis_index("c")
        s = lax.axis_index("s")
        wid = s * NC + c
        r0 = s * RPT
        last_tile = s == NS - 1

        pltpu.sync_copy(edges.at[0, wid], src_v)
        pltpu.sync_copy(edges.at[1, wid], dst_v)
        if with_deg:
            pltpu.sync_copy(ones_h, ones_v)
            pltpu.sync_copy(zdeg.at[pl.ds(r0, RPT)], dacc.at[pl.ds(r0, RPT)])

            @pl.when(last_tile)
            def _():
                pltpu.sync_copy(zdeg.at[pl.ds(TAIL0, TAIL)],
                                dacc.at[pl.ds(TAIL0, TAIL)])

        for p in range(P):
            y, out = tables[p], outs[p]
            deg_here = with_deg and p == 0

            pltpu.sync_copy(zrows.at[pl.ds(r0, RPT)], acc.at[pl.ds(r0, RPT)])

            @pl.when(last_tile)
            def _():
                pltpu.sync_copy(zrows.at[pl.ds(TAIL0, TAIL)],
                                acc.at[pl.ds(TAIL0, TAIL)])

            plsc.subcore_barrier()

            def gather(i, b):
                pltpu.async_copy(y.at[src_v.at[i]], bufs[b], sems[b])

            def slot(i, b):
                pltpu.make_async_copy(y.at[src_v.at[i]], bufs[b],
                                      sems[b]).wait()
                pltpu.sync_copy(bufs[b], acc.at[dst_v.at[i]], add=True)
                if deg_here:
                    pltpu.async_copy(ones_v, dacc.at[dst_v.at[i]], dsem,
                                     add=True)

                @pl.when(i + NBUF < NCHUNK)
                def _():
                    gather(i + NBUF, b)

                if deg_here:
                    pltpu.make_async_copy(ones_v, dacc.at[dst_v.at[i]],
                                          dsem).wait()

            for b in range(NBUF):
                gather(b, b)

            def turn(t, carry):
                for b in range(NBUF):
                    slot(t * NBUF + b, b)
                return carry

            lax.fori_loop(0, NCHUNK // NBUF, turn, 0)
            for r in range(NCHUNK % NBUF):
                slot(NBUF * (NCHUNK // NBUF) + r, r)
            plsc.subcore_barrier()

            pltpu.sync_copy(acc.at[pl.ds(r0, RPT)], out.at[c, pl.ds(r0, RPT)])
            if deg_here:
                pltpu.sync_copy(dacc.at[pl.ds(r0, RPT)],
                                deg_out.at[c, pl.ds(r0, RPT)])

            @pl.when(last_tile)
            def _flush_tail():
                pltpu.sync_copy(acc.at[pl.ds(TAIL0, TAIL)],
                                out.at[c, pl.ds(TAIL0, TAIL)])
                if deg_here:
                    pltpu.sync_copy(dacc.at[pl.ds(TAIL0, TAIL)],
                                    deg_out.at[c, pl.ds(TAIL0, TAIL)])

    return pl.kernel(body, out_type=out_type, mesh=mesh,
                     scratch_types=scratch,
                     compiler_params=pltpu.CompilerParams(
                         use_tc_tiling_on_sc=False))


_BM = 1000


def _tc1_body(x, wl, wr, bl, y1, r1b):
    xv = x[...]
    y1[...] = jnp.dot(xv, wl[...], preferred_element_type=jnp.float32)
    r1b[...] = jnp.dot(xv, wr[...], preferred_element_type=jnp.float32) + bl[...]


def _tc2a_body(a1, degp, r1b, wl2, y2, h_out, deg):
    agg = a1[0] + a1[1]
    d = jnp.maximum(degp[0] + degp[1], 1.0)
    pre = agg / d[:, 0:1] + r1b[...]
    h = jnp.where(pre > 0, pre, jnp.exp(jnp.minimum(pre, 0.0)) - 1.0)
    y2[...] = jnp.dot(h, wl2[...], preferred_element_type=jnp.float32)
    h_out[...] = h
    deg[...] = d


def _tc2b_body(h, wr2, bl2, r2b):
    r2b[...] = jnp.dot(h[...], wr2[...], preferred_element_type=jnp.float32) + bl2[...]


def _tc3_body(a2, deg, r2b, out):
    out[...] = (a2[0, :, :C] + a2[1, :, :C]) / deg[:, 0:1] + r2b[...]


def kernel(x, edge_index, Wl1, bl1, Wr1, Wl2, bl2, Wr2, Q, epoch):
    e4 = edge_index.reshape(2, NW, NCHUNK, K)
    f32 = jnp.float32

    nblk = N // _BM
    row_spec = lambda w: pl.BlockSpec((_BM, w), lambda i: (i, 0))
    full = lambda shape: pl.BlockSpec(shape, lambda i: tuple(0 for _ in shape))
    part_spec = lambda w: pl.BlockSpec((NC, _BM, w), lambda i: (0, i, 0))

    y1, r1b = pl.pallas_call(
        _tc1_body,
        grid=(nblk,),
        in_specs=[row_spec(F), full((F, H)), full((F, H)), full((1, H))],
        out_specs=[row_spec(H), row_spec(H)],
        out_shape=[jax.ShapeDtypeStruct((N, H), f32),
                   jax.ShapeDtypeStruct((N, H), f32)],
    )(x, Wl1, Wr1, bl1.reshape(1, H))

    zrows = jnp.zeros((N, H), f32)
    zdeg = jnp.zeros((N, DW), f32)
    ones_h = jnp.ones((K, DW), f32)
    a1, degp = _make_agg(H, True, 5, 1)(y1, e4, zrows, zdeg, ones_h)

    Wl2p = jnp.pad(Wl2, ((0, 0), (0, CP - C)))
    y2, h, deg = pl.pallas_call(
        _tc2a_body,
        grid=(nblk,),
        in_specs=[part_spec(H), part_spec(DW), row_spec(H), full((H, CP))],
        out_specs=[row_spec(CP), row_spec(H), row_spec(DW)],
        out_shape=[jax.ShapeDtypeStruct((N, CP), f32),
                   jax.ShapeDtypeStruct((N, H), f32),
                   jax.ShapeDtypeStruct((N, DW), f32)],
    )(a1, degp, r1b, Wl2p)

    r2b = pl.pallas_call(
        _tc2b_body,
        grid=(nblk,),
        in_specs=[row_spec(H), full((H, C)), full((1, C))],
        out_specs=row_spec(C),
        out_shape=jax.ShapeDtypeStruct((N, C), f32),
    )(h, Wr2, bl2.reshape(1, C))

    zrows2 = jnp.zeros((N, CP), f32)
    a2 = _make_agg(CP, False, 12, 1)(y2, e4, zrows2)

    out = pl.pallas_call(
        _tc3_body,
        grid=(nblk,),
        in_specs=[part_spec(CP), row_spec(DW), row_spec(C)],
        out_specs=row_spec(C),
        out_shape=jax.ShapeDtypeStruct((N, C), f32),
    )(a2, deg, r2b)

    return (out, Q)


# speedup vs baseline: 1.1347x; 1.0002x over previous
import functools

import jax
import jax.numpy as jnp
from jax import lax
from jax.experimental import pallas as pl
from jax.experimental.pallas import tpu as pltpu
from jax.experimental.pallas import tpu_sc as plsc

N = 10000
E = 320000
F = 128
H = 128
C = 40
CP = 48

NC = 2
NS = 16
NW = NC * NS
EW = E // NW
K = 40
NCHUNK = EW // K
DW = 8
RPT = 624
TAIL0 = NS * RPT
TAIL = N - TAIL0


@functools.lru_cache(maxsize=None)
def _make_agg(D, with_deg, NBUF, P):
    mesh = plsc.VectorSubcoreMesh(
        core_axis_name="c", subcore_axis_name="s",
        num_cores=NC, num_subcores=NS)
    parts = [jax.ShapeDtypeStruct((NC, N, D), jnp.float32)] * P
    if with_deg:
        parts.append(jax.ShapeDtypeStruct((NC, N, DW), jnp.float32))
    out_type = tuple(parts) if len(parts) > 1 else parts[0]
    scratch = [
        pltpu.VMEM((NCHUNK, K), jnp.int32),
        pltpu.VMEM((NCHUNK, K), jnp.int32),
    ] + [pltpu.VMEM((K, D), jnp.float32) for _ in range(NBUF)] + [
        pltpu.VMEM_SHARED((N, D), jnp.float32),
    ] + [pltpu.SemaphoreType.DMA for _ in range(NBUF + 1)]
    if with_deg:
        scratch += [
            pltpu.VMEM((K, DW), jnp.float32),
            pltpu.VMEM_SHARED((N, DW), jnp.float32),
        ]

    def body(*refs):
        tables = refs[:P]
        edges, zrows = refs[P], refs[P + 1]
        nin = P + 2
        if with_deg:
            zdeg, ones_h = refs[P + 2], refs[P + 3]
            nin = P + 4
        outs = refs[nin:nin + P]
        deg_out = refs[nin + P] if with_deg else None
        rest = refs[nin + P + (1 if with_deg else 0):]
        src_v, dst_v = rest[0], rest[1]
        bufs = rest[2:2 + NBUF]
        acc = rest[2 + NBUF]
        sems = rest[3 + NBUF:3 + 2 * NBUF]
        dsem = rest[3 + 2 * NBUF]
        if with_deg:
            ones_v, dacc = rest[4 + 2 * NBUF], rest[5 + 2 * NBUF]

        c = lax.axis_index("c")
        s = lax.axis_index("s")
        wid = s * NC + c
        r0 = s * RPT
        last_tile = s == NS - 1

        pltpu.sync_copy(edges.at[0, wid], src_v)
        pltpu.sync_copy(edges.at[1, wid], dst_v)
        if with_deg:
            pltpu.sync_copy(ones_h, ones_v)
            pltpu.sync_copy(zdeg.at[pl.ds(r0, RPT)], dacc.at[pl.ds(r0, RPT)])

            @pl.when(last_tile)
            def _():
                pltpu.sync_copy(zdeg.at[pl.ds(TAIL0, TAIL)],
                                dacc.at[pl.ds(TAIL0, TAIL)])

        for p in range(P):
            y, out = tables[p], outs[p]
            deg_here = with_deg and p == 0

            pltpu.sync_copy(zrows.at[pl.ds(r0, RPT)], acc.at[pl.ds(r0, RPT)])

            @pl.when(last_tile)
            def _():
                pltpu.sync_copy(zrows.at[pl.ds(TAIL0, TAIL)],
                                acc.at[pl.ds(TAIL0, TAIL)])

            plsc.subcore_barrier()

            def gather(i, b):
                pltpu.async_copy(y.at[src_v.at[i]], bufs[b], sems[b])

            def slot(i, b):
                pltpu.make_async_copy(y.at[src_v.at[i]], bufs[b],
                                      sems[b]).wait()
                pltpu.sync_copy(bufs[b], acc.at[dst_v.at[i]], add=True)
                if deg_here:
                    pltpu.async_copy(ones_v, dacc.at[dst_v.at[i]], dsem,
                                     add=True)

                @pl.when(i + NBUF < NCHUNK)
                def _():
                    gather(i + NBUF, b)

                if deg_here:
                    pltpu.make_async_copy(ones_v, dacc.at[dst_v.at[i]],
                                          dsem).wait()

            for b in range(NBUF):
                gather(b, b)

            def turn(t, carry):
                for b in range(NBUF):
                    slot(t * NBUF + b, b)
                return carry

            lax.fori_loop(0, NCHUNK // NBUF, turn, 0)
            for r in range(NCHUNK % NBUF):
                slot(NBUF * (NCHUNK // NBUF) + r, r)
            plsc.subcore_barrier()

            pltpu.sync_copy(acc.at[pl.ds(r0, RPT)], out.at[c, pl.ds(r0, RPT)])
            if deg_here:
                pltpu.sync_copy(dacc.at[pl.ds(r0, RPT)],
                                deg_out.at[c, pl.ds(r0, RPT)])

            @pl.when(last_tile)
            def _flush_tail():
                pltpu.sync_copy(acc.at[pl.ds(TAIL0, TAIL)],
                                out.at[c, pl.ds(TAIL0, TAIL)])
                if deg_here:
                    pltpu.sync_copy(dacc.at[pl.ds(TAIL0, TAIL)],
                                    deg_out.at[c, pl.ds(TAIL0, TAIL)])

    return pl.kernel(body, out_type=out_type, mesh=mesh,
                     scratch_types=scratch,
                     compiler_params=pltpu.CompilerParams(
                         use_tc_tiling_on_sc=False))


_BM = 1000


def _tc1_body(x, wl, wr, bl, y1, r1b):
    xv = x[...]
    y1[...] = jnp.dot(xv, wl[...], preferred_element_type=jnp.float32)
    r1b[...] = jnp.dot(xv, wr[...], preferred_element_type=jnp.float32) + bl[...]


def _tc2a_body(a1, degp, r1b, wl2, y2, h_out, deg):
    agg = a1[0] + a1[1]
    d = jnp.maximum(degp[0] + degp[1], 1.0)
    pre = agg / d[:, 0:1] + r1b[...]
    h = jnp.where(pre > 0, pre, jnp.exp(jnp.minimum(pre, 0.0)) - 1.0)
    y2[...] = jnp.dot(h, wl2[...], preferred_element_type=jnp.float32)
    h_out[...] = h
    deg[...] = d


def _tc2b_body(h, wr2, bl2, r2b):
    r2b[...] = jnp.dot(h[...], wr2[...], preferred_element_type=jnp.float32) + bl2[...]


def _tc3_body(a2, deg, r2b, out):
    out[...] = (a2[0, :, :C] + a2[1, :, :C]) / deg[:, 0:1] + r2b[...]


def kernel(x, edge_index, Wl1, bl1, Wr1, Wl2, bl2, Wr2, Q, epoch):
    e4 = edge_index.reshape(2, NW, NCHUNK, K)
    f32 = jnp.float32

    nblk = N // _BM
    row_spec = lambda w: pl.BlockSpec((_BM, w), lambda i: (i, 0))
    full = lambda shape: pl.BlockSpec(shape, lambda i: tuple(0 for _ in shape))
    part_spec = lambda w: pl.BlockSpec((NC, _BM, w), lambda i: (0, i, 0))

    y1, r1b = pl.pallas_call(
        _tc1_body,
        grid=(nblk,),
        in_specs=[row_spec(F), full((F, H)), full((F, H)), full((1, H))],
        out_specs=[row_spec(H), row_spec(H)],
        out_shape=[jax.ShapeDtypeStruct((N, H), f32),
                   jax.ShapeDtypeStruct((N, H), f32)],
    )(x, Wl1, Wr1, bl1.reshape(1, H))

    zrows = jnp.zeros((N, H), f32)
    zdeg = jnp.zeros((N, DW), f32)
    ones_h = jnp.ones((K, DW), f32)
    a1, degp = _make_agg(H, True, 5, 1)(y1, e4, zrows, zdeg, ones_h)
    Qc = Q + jnp.float32(0.0)
    a1, Qc = lax.optimization_barrier((a1, Qc))

    Wl2p = jnp.pad(Wl2, ((0, 0), (0, CP - C)))
    y2, h, deg = pl.pallas_call(
        _tc2a_body,
        grid=(nblk,),
        in_specs=[part_spec(H), part_spec(DW), row_spec(H), full((H, CP))],
        out_specs=[row_spec(CP), row_spec(H), row_spec(DW)],
        out_shape=[jax.ShapeDtypeStruct((N, CP), f32),
                   jax.ShapeDtypeStruct((N, H), f32),
                   jax.ShapeDtypeStruct((N, DW), f32)],
    )(a1, degp, r1b, Wl2p)

    r2b = pl.pallas_call(
        _tc2b_body,
        grid=(nblk,),
        in_specs=[row_spec(H), full((H, C)), full((1, C))],
        out_specs=row_spec(C),
        out_shape=jax.ShapeDtypeStruct((N, C), f32),
    )(h, Wr2, bl2.reshape(1, C))

    zrows2 = jnp.zeros((N, CP), f32)
    a2 = _make_agg(CP, False, 12, 1)(y2, e4, zrows2)

    out = pl.pallas_call(
        _tc3_body,
        grid=(nblk,),
        in_specs=[part_spec(CP), row_spec(DW), row_spec(C)],
        out_specs=row_spec(C),
        out_shape=jax.ShapeDtypeStruct((N, C), f32),
    )(a2, deg, r2b)

    return (out, Qc)


# speedup vs baseline: 1.1362x; 1.0013x over previous
import functools

import jax
import jax.numpy as jnp
from jax import lax
from jax.experimental import pallas as pl
from jax.experimental.pallas import tpu as pltpu
from jax.experimental.pallas import tpu_sc as plsc

N = 10000
E = 320000
F = 128
H = 128
C = 40
CP = 48

NC = 2
NS = 16
NW = NC * NS
EW = E // NW
K = 40
NCHUNK = EW // K
DW = 8
RPT = 624
TAIL0 = NS * RPT
TAIL = N - TAIL0


@functools.lru_cache(maxsize=None)
def _make_agg(D, with_deg, NBUF, P):
    mesh = plsc.VectorSubcoreMesh(
        core_axis_name="c", subcore_axis_name="s",
        num_cores=NC, num_subcores=NS)
    parts = [jax.ShapeDtypeStruct((NC, N, D), jnp.float32)] * P
    if with_deg:
        parts.append(jax.ShapeDtypeStruct((NC, N, DW), jnp.float32))
    out_type = tuple(parts) if len(parts) > 1 else parts[0]
    scratch = [
        pltpu.VMEM((NCHUNK, K), jnp.int32),
        pltpu.VMEM((NCHUNK, K), jnp.int32),
    ] + [pltpu.VMEM((K, D), jnp.float32) for _ in range(NBUF)] + [
        pltpu.VMEM_SHARED((N, D), jnp.float32),
    ] + [pltpu.SemaphoreType.DMA for _ in range(NBUF + 1)]
    if with_deg:
        scratch += [
            pltpu.VMEM((K, DW), jnp.float32),
            pltpu.VMEM_SHARED((N, DW), jnp.float32),
        ]

    def body(*refs):
        tables = refs[:P]
        edges, zrows = refs[P], refs[P + 1]
        nin = P + 2
        if with_deg:
            zdeg, ones_h = refs[P + 2], refs[P + 3]
            nin = P + 4
        outs = refs[nin:nin + P]
        deg_out = refs[nin + P] if with_deg else None
        rest = refs[nin + P + (1 if with_deg else 0):]
        src_v, dst_v = rest[0], rest[1]
        bufs = rest[2:2 + NBUF]
        acc = rest[2 + NBUF]
        sems = rest[3 + NBUF:3 + 2 * NBUF]
        dsem = rest[3 + 2 * NBUF]
        if with_deg:
            ones_v, dacc = rest[4 + 2 * NBUF], rest[5 + 2 * NBUF]

        c = lax.axis_index("c")
        s = lax.axis_index("s")
        wid = s * NC + c
        r0 = s * RPT
        last_tile = s == NS - 1

        pltpu.sync_copy(edges.at[0, wid], src_v)
        pltpu.sync_copy(edges.at[1, wid], dst_v)
        if with_deg:
            pltpu.sync_copy(ones_h, ones_v)
            pltpu.sync_copy(zdeg.at[pl.ds(r0, RPT)], dacc.at[pl.ds(r0, RPT)])

            @pl.when(last_tile)
            def _():
                pltpu.sync_copy(zdeg.at[pl.ds(TAIL0, TAIL)],
                                dacc.at[pl.ds(TAIL0, TAIL)])

        for p in range(P):
            y, out = tables[p], outs[p]
            deg_here = with_deg and p == 0

            pltpu.sync_copy(zrows.at[pl.ds(r0, RPT)], acc.at[pl.ds(r0, RPT)])

            @pl.when(last_tile)
            def _():
                pltpu.sync_copy(zrows.at[pl.ds(TAIL0, TAIL)],
                                acc.at[pl.ds(TAIL0, TAIL)])

            plsc.subcore_barrier()

            def gather(i, b):
                pltpu.async_copy(y.at[src_v.at[i]], bufs[b], sems[b])

            def slot(i, b):
                pltpu.make_async_copy(y.at[src_v.at[i]], bufs[b],
                                      sems[b]).wait()
                pltpu.sync_copy(bufs[b], acc.at[dst_v.at[i]], add=True)
                if deg_here:
                    pltpu.async_copy(ones_v, dacc.at[dst_v.at[i]], dsem,
                                     add=True)

                @pl.when(i + NBUF < NCHUNK)
                def _():
                    gather(i + NBUF, b)

                if deg_here:
                    pltpu.make_async_copy(ones_v, dacc.at[dst_v.at[i]],
                                          dsem).wait()

            for b in range(NBUF):
                gather(b, b)

            def turn(t, carry):
                for b in range(NBUF):
                    slot(t * NBUF + b, b)
                return carry

            lax.fori_loop(0, NCHUNK // NBUF, turn, 0)
            for r in range(NCHUNK % NBUF):
                slot(NBUF * (NCHUNK // NBUF) + r, r)
            plsc.subcore_barrier()

            pltpu.sync_copy(acc.at[pl.ds(r0, RPT)], out.at[c, pl.ds(r0, RPT)])
            if deg_here:
                pltpu.sync_copy(dacc.at[pl.ds(r0, RPT)],
                                deg_out.at[c, pl.ds(r0, RPT)])

            @pl.when(last_tile)
            def _flush_tail():
                pltpu.sync_copy(acc.at[pl.ds(TAIL0, TAIL)],
                                out.at[c, pl.ds(TAIL0, TAIL)])
                if deg_here:
                    pltpu.sync_copy(dacc.at[pl.ds(TAIL0, TAIL)],
                                    deg_out.at[c, pl.ds(TAIL0, TAIL)])

    return pl.kernel(body, out_type=out_type, mesh=mesh,
                     scratch_types=scratch,
                     compiler_params=pltpu.CompilerParams(
                         use_tc_tiling_on_sc=False))


_BM = 1000


def _tc1_body(x, wl, wr, bl, y1, r1b):
    xv = x[...]
    y1[...] = jnp.dot(xv, wl[...], preferred_element_type=jnp.float32)
    r1b[...] = jnp.dot(xv, wr[...], preferred_element_type=jnp.float32) + bl[...]


def _tc2a_body(a1, degp, r1b, wl2, y2, h_out, deg):
    agg = a1[0] + a1[1]
    d = jnp.maximum(degp[0] + degp[1], 1.0)
    pre = agg / d[:, 0:1] + r1b[...]
    h = jnp.where(pre > 0, pre, jnp.exp(jnp.minimum(pre, 0.0)) - 1.0)
    y2[...] = jnp.dot(h, wl2[...], preferred_element_type=jnp.float32)
    h_out[...] = h
    deg[...] = d


def _tc2b_body(h, wr2, bl2, r2b):
    r2b[...] = jnp.dot(h[...], wr2[...], preferred_element_type=jnp.float32) + bl2[...]


def _tc3_body(a2, deg, r2b, out):
    out[...] = (a2[0, :, :C] + a2[1, :, :C]) / deg[:, 0:1] + r2b[...]


def kernel(x, edge_index, Wl1, bl1, Wr1, Wl2, bl2, Wr2, Q, epoch):
    e4 = edge_index.reshape(2, NW, NCHUNK, K)
    f32 = jnp.float32

    nblk = N // _BM
    row_spec = lambda w: pl.BlockSpec((_BM, w), lambda i: (i, 0))
    full = lambda shape: pl.BlockSpec(shape, lambda i: tuple(0 for _ in shape))
    part_spec = lambda w: pl.BlockSpec((NC, _BM, w), lambda i: (0, i, 0))

    y1, r1b = pl.pallas_call(
        _tc1_body,
        grid=(nblk,),
        in_specs=[row_spec(F), full((F, H)), full((F, H)), full((1, H))],
        out_specs=[row_spec(H), row_spec(H)],
        out_shape=[jax.ShapeDtypeStruct((N, H), f32),
                   jax.ShapeDtypeStruct((N, H), f32)],
    )(x, Wl1, Wr1, bl1.reshape(1, H))

    zrows = jnp.zeros((N, H), f32)
    zdeg = jnp.zeros((N, DW), f32)
    ones_h = jnp.ones((K, DW), f32)
    a1, degp = _make_agg(H, True, 5, 1)(y1, e4, zrows, zdeg, ones_h)

    Wl2p = jnp.pad(Wl2, ((0, 0), (0, CP - C)))
    y2, h, deg = pl.pallas_call(
        _tc2a_body,
        grid=(nblk,),
        in_specs=[part_spec(H), part_spec(DW), row_spec(H), full((H, CP))],
        out_specs=[row_spec(CP), row_spec(H), row_spec(DW)],
        out_shape=[jax.ShapeDtypeStruct((N, CP), f32),
                   jax.ShapeDtypeStruct((N, H), f32),
                   jax.ShapeDtypeStruct((N, DW), f32)],
    )(a1, degp, r1b, Wl2p)

    r2b = pl.pallas_call(
        _tc2b_body,
        grid=(nblk,),
        in_specs=[row_spec(H), full((H, C)), full((1, C))],
        out_specs=row_spec(C),
        out_shape=jax.ShapeDtypeStruct((N, C), f32),
    )(h, Wr2, bl2.reshape(1, C))

    zrows2 = jnp.zeros((N, CP), f32)
    a2 = _make_agg(CP, False, 12, 1)(y2, e4, zrows2)

    out = pl.pallas_call(
        _tc3_body,
        grid=(nblk,),
        in_specs=[part_spec(CP), row_spec(DW), row_spec(C)],
        out_specs=row_spec(C),
        out_shape=jax.ShapeDtypeStruct((N, C), f32),
    )(a2, deg, r2b)

    return (out, Q)
